# baseline (speedup 1.0000x reference)
import jax
import jax.numpy as jnp
from jax.experimental import pallas as pl
from jax.experimental.pallas import tpu as pltpu

_R = 0.02

_NY, _NX = 4096, 16384
_BY = 256


def _dist_kernel(y_ref, xt_ref, masked_ref, counts_ref):
    y = y_ref[...]
    y0 = y[:, 0]
    y1 = y[:, 1]
    yn = y0 * y0 + y1 * y1
    x0 = xt_ref[0, :]
    x1 = xt_ref[1, :]
    xn = x0 * x0 + x1 * x1
    ymb = (y * (-2.0)).astype(jnp.bfloat16)
    xbb = xt_ref[...].astype(jnp.bfloat16)
    dotm2 = jnp.dot(ymb, xbb, preferred_element_type=jnp.float32)
    d2 = (yn[:, None] + xn[None, :]) + dotm2
    dist = jnp.sqrt(jnp.maximum(d2, 0.0))
    mask = dist <= _R
    masked_ref[...] = jnp.where(mask, dist, 0.0)
    counts_ref[...] = jnp.sum(mask.astype(jnp.int32), axis=1)[:, None]


def kernel(x, y):
    xt = x.T
    masked, counts = pl.pallas_call(
        _dist_kernel,
        grid=(_NY // _BY,),
        in_specs=[
            pl.BlockSpec((_BY, 2), lambda i: (i, 0)),
            pl.BlockSpec((2, _NX), lambda i: (0, 0)),
        ],
        out_specs=[
            pl.BlockSpec((_BY, _NX), lambda i: (i, 0)),
            pl.BlockSpec((_BY, 1), lambda i: (i, 0)),
        ],
        out_shape=[
            jax.ShapeDtypeStruct((_NY, _NX), jnp.float32),
            jax.ShapeDtypeStruct((_NY, 1), jnp.int32),
        ],
        compiler_params=pltpu.CompilerParams(
            dimension_semantics=("parallel",),
        ),
    )(y, xt)
    return masked, counts.reshape(_NY)


# speedup vs baseline: 1.0036x; 1.0036x over previous
import jax
import jax.numpy as jnp
from jax.experimental import pallas as pl
from jax.experimental.pallas import tpu as pltpu

_R = 0.02

_NY, _NX = 4096, 16384
_BY, _BX = 1024, 2048


def _dist_kernel(y_ref, xt_ref, masked_ref, counts_ref):
    y = y_ref[...]
    y0 = y[:, 0]
    y1 = y[:, 1]
    yn = y0 * y0 + y1 * y1
    x0 = xt_ref[0, :]
    x1 = xt_ref[1, :]
    xn = x0 * x0 + x1 * x1
    ymb = (y * (-2.0)).astype(jnp.bfloat16)
    xbb = xt_ref[...].astype(jnp.bfloat16)
    dotm2 = jnp.dot(ymb, xbb, preferred_element_type=jnp.float32)
    d2 = (yn[:, None] + xn[None, :]) + dotm2
    dist = jnp.sqrt(jnp.maximum(d2, 0.0))
    mask = dist <= _R
    masked_ref[...] = jnp.where(mask, dist, 0.0)
    partial = jnp.sum(mask.astype(jnp.int32), axis=1)[:, None]

    j = pl.program_id(1)

    @pl.when(j == 0)
    def _init():
        counts_ref[...] = partial

    @pl.when(j != 0)
    def _acc():
        counts_ref[...] += partial


def kernel(x, y):
    xt = x.T
    masked, counts = pl.pallas_call(
        _dist_kernel,
        grid=(_NY // _BY, _NX // _BX),
        in_specs=[
            pl.BlockSpec((_BY, 2), lambda i, j: (i, 0)),
            pl.BlockSpec((2, _BX), lambda i, j: (0, j)),
        ],
        out_specs=[
            pl.BlockSpec((_BY, _BX), lambda i, j: (i, j)),
            pl.BlockSpec((_BY, 1), lambda i, j: (i, 0)),
        ],
        out_shape=[
            jax.ShapeDtypeStruct((_NY, _NX), jnp.float32),
            jax.ShapeDtypeStruct((_NY, 1), jnp.int32),
        ],
        compiler_params=pltpu.CompilerParams(
            dimension_semantics=("parallel", "arbitrary"),
        ),
    )(y, xt)
    return masked, counts.reshape(_NY)


# speedup vs baseline: 1.0624x; 1.0585x over previous
import jax
import jax.numpy as jnp
from jax.experimental import pallas as pl
from jax.experimental.pallas import tpu as pltpu

_R = 0.02

_NY, _NX = 4096, 16384
_BY, _BX = 1024, 4096


def _dist_kernel(y_ref, xt_ref, masked_ref, counts_ref):
    y = y_ref[...]
    y0 = y[:, 0]
    y1 = y[:, 1]
    yn = y0 * y0 + y1 * y1
    x0 = xt_ref[0, :]
    x1 = xt_ref[1, :]
    xn = x0 * x0 + x1 * x1
    ymb = (y * (-2.0)).astype(jnp.bfloat16)
    xbb = xt_ref[...].astype(jnp.bfloat16)
    dotm2 = jnp.dot(ymb, xbb, preferred_element_type=jnp.float32)
    d2 = (yn[:, None] + xn[None, :]) + dotm2
    dist = jnp.sqrt(jnp.maximum(d2, 0.0))
    mask = dist <= _R
    masked_ref[...] = jnp.where(mask, dist, 0.0)
    partial = jnp.sum(mask.astype(jnp.int32), axis=1)[:, None]

    j = pl.program_id(1)

    @pl.when(j == 0)
    def _init():
        counts_ref[...] = partial

    @pl.when(j != 0)
    def _acc():
        counts_ref[...] += partial


def kernel(x, y):
    xt = x.T
    masked, counts = pl.pallas_call(
        _dist_kernel,
        grid=(_NY // _BY, _NX // _BX),
        in_specs=[
            pl.BlockSpec((_BY, 2), lambda i, j: (i, 0)),
            pl.BlockSpec((2, _BX), lambda i, j: (0, j)),
        ],
        out_specs=[
            pl.BlockSpec((_BY, _BX), lambda i, j: (i, j)),
            pl.BlockSpec((_BY, 1), lambda i, j: (i, 0)),
        ],
        out_shape=[
            jax.ShapeDtypeStruct((_NY, _NX), jnp.float32),
            jax.ShapeDtypeStruct((_NY, 1), jnp.int32),
        ],
        compiler_params=pltpu.CompilerParams(
            dimension_semantics=("parallel", "arbitrary"),
        ),
    )(y, xt)
    return masked, counts.reshape(_NY)
